# baseline (speedup 1.0000x reference)
import functools

import jax
import jax.numpy as jnp
from jax import lax
from jax.experimental import pallas as pl
from jax.experimental.pallas import tpu as pltpu
from jax.experimental.pallas import tpu_sc as plsc

_B = 4096
_D = 1024
_S = 64
_JPAD = 80
_NC, _NS = 2, 16
_NW = _NC * _NS
_RPW = _B // _NW
_G = 16
_NGRP = _RPW // _G
_DC = _D // 16
_NBUF = 4
_UNR = 1


def _sc_scores_body(x_hbm, valid_hbm, samp_hbm, w_hbm, out_hbm,
                    xg2, samp_v2, valid_v2, buf, tbuf, scores_v, sems, sema, semst,
                    semsc):
    wid = lax.axis_index("s") * _NC + lax.axis_index("c")
    iota = lax.iota(jnp.int32, 16)

    def reduce16(accs):
        for r in range(16):
            tbuf[pl.ds(r * 16, 16)] = accs[r]
        vec = plsc.load_gather(tbuf, [iota * 16])
        for l in range(1, 16):
            vec = vec + plsc.load_gather(tbuf, [iota * 16 + l])
        return vec

    zeros16 = tuple(jnp.zeros((16,), jnp.float32) for _ in range(16))

    def stage_start(g):
        p = g % 2
        b0 = wid * _RPW + g * _G
        pltpu.async_copy(x_hbm.at[pl.ds(b0, _G)], xg2.at[p], semst)
        pltpu.async_copy(samp_hbm.at[pl.ds(b0 * 4, 64)], samp_v2.at[p], semst)
        pltpu.async_copy(valid_hbm.at[pl.ds(b0, _G)], valid_v2.at[p], semst)

    def stage_wait(g):
        p = g % 2
        b0 = wid * _RPW + g * _G
        pltpu.make_async_copy(x_hbm.at[pl.ds(b0, _G)], xg2.at[p], semst).wait()
        pltpu.make_async_copy(samp_hbm.at[pl.ds(b0 * 4, 64)], samp_v2.at[p],
                              semst).wait()
        pltpu.make_async_copy(valid_hbm.at[pl.ds(b0, _G)], valid_v2.at[p],
                              semst).wait()

    stage_start(0)
    stage_wait(0)
    pltpu.async_copy(w_hbm.at[valid_v2.at[0]], buf.at[0], sema)

    def group(g, _):
        p = g % 2
        xg = xg2.at[p]
        samp_v = samp_v2.at[p]
        valid_v = valid_v2.at[p]
        b0 = wid * _RPW + g * _G

        def start(cc, nb):
            pltpu.async_copy(w_hbm.at[samp_v.at[cc]], buf.at[nb], sems.at[nb])

        for nb in range(1, _NBUF):
            start(nb, nb)

        @pl.when(g + 1 < _NGRP)
        def _():
            stage_start(g + 1)

        pltpu.make_async_copy(w_hbm.at[valid_v], buf.at[0], sema).wait()

        def dstep_a(k, accs):
            k16 = k * _UNR * 16
            for u in range(_UNR):
                ku = k16 + u * 16
                accs = tuple(
                    accs[r] + buf[0, r, pl.ds(ku, 16)] * xg[r, pl.ds(ku, 16)]
                    for r in range(16))
            return accs

        s0 = reduce16(lax.fori_loop(0, _DC // _UNR, dstep_a, zeros16))

        @pl.when(g > 0)
        def _():
            pltpu.make_async_copy(
                scores_v, out_hbm.at[pl.ds(b0, _G)], semsc).wait()

        plsc.store_scatter(scores_v, [iota, iota * 0], s0)
        start(0, 0)

        def quad(t, _):
            for nb in range(_NBUF):
                c = t * _NBUF + nb
                pltpu.make_async_copy(w_hbm.at[samp_v.at[c]], buf.at[nb],
                                      sems.at[nb]).wait()
                bl = c // 4

                def dstep_b(k, accs, nb=nb, bl=bl):
                    k16 = k * _UNR * 16
                    for u in range(_UNR):
                        ku = k16 + u * 16
                        xc = xg[bl, pl.ds(ku, 16)]
                        accs = tuple(
                            accs[r] + buf[nb, r, pl.ds(ku, 16)] * xc
                            for r in range(16))
                    return accs

                vec = reduce16(lax.fori_loop(0, _DC // _UNR, dstep_b, zeros16))
                scores_v[bl, pl.ds(1 + (c % 4) * 16, 16)] = vec

                @pl.when(c + _NBUF < 64)
                def _(c=c, nb=nb):
                    start(c + _NBUF, nb)
            return 0

        lax.fori_loop(0, 64 // _NBUF, quad, 0)

        @pl.when(g + 1 < _NGRP)
        def _():
            stage_wait(g + 1)
            pltpu.async_copy(w_hbm.at[valid_v2.at[(g + 1) % 2]], buf.at[0],
                             sema)

        pltpu.async_copy(scores_v, out_hbm.at[pl.ds(b0, _G)], semsc)
        return 0

    lax.fori_loop(0, _NGRP, group, 0)
    pltpu.make_async_copy(
        scores_v, out_hbm.at[pl.ds(0, _G)], semsc).wait()


_sc_scores = functools.partial(
    pl.kernel,
    out_type=jax.ShapeDtypeStruct((_B, _JPAD), jnp.float32),
    mesh=plsc.VectorSubcoreMesh(core_axis_name="c", subcore_axis_name="s",
                                num_cores=_NC, num_subcores=_NS),
    scratch_types=[
        pltpu.VMEM((2, _G, _D), jnp.float32),
        pltpu.VMEM((2, 64, 16), jnp.int32),
        pltpu.VMEM((2, 16), jnp.int32),
        pltpu.VMEM((_NBUF, 16, _D), jnp.float32),
        pltpu.VMEM((256,), jnp.float32),
        pltpu.VMEM((_G, _JPAD), jnp.float32),
        pltpu.SemaphoreType.DMA((_NBUF,)),
        pltpu.SemaphoreType.DMA,
        pltpu.SemaphoreType.DMA,
        pltpu.SemaphoreType.DMA,
    ],
    compiler_params=pltpu.CompilerParams(needs_layout_passes=False),
)(_sc_scores_body)


def _loss_body(s_ref, o_ref):
    s = s_ref[...]
    lane = lax.broadcasted_iota(jnp.int32, (_B, _JPAD), 1)
    mask = lane < (1 + _S)
    sm = jnp.where(mask, s, -1e30)
    m = jnp.max(sm, axis=1, keepdims=True)
    e = jnp.where(mask, jnp.exp(sm - m), 0.0)
    p = e / jnp.sum(e, axis=1, keepdims=True)
    e2 = jnp.where(mask, jnp.exp(p), 0.0)
    lse = jnp.log(jnp.sum(e2, axis=1, keepdims=True))
    p0 = jnp.sum(jnp.where(lane == 0, p, 0.0), axis=1, keepdims=True)
    o_ref[...] = (jnp.sum(lse - p0) / _B).reshape(1, 1)


def kernel(x, valid_indices, samples_indices, W):
    valid = valid_indices.astype(jnp.int32)
    samp = samples_indices.astype(jnp.int32).reshape(_B * 4, 16)
    scores = _sc_scores(x, valid, samp, W)
    loss = pl.pallas_call(
        _loss_body,
        out_shape=jax.ShapeDtypeStruct((1, 1), jnp.float32),
    )(scores)
    return loss[0, 0]


# speedup vs baseline: 1.0085x; 1.0085x over previous
import functools

import jax
import jax.numpy as jnp
from jax import lax
from jax.experimental import pallas as pl
from jax.experimental.pallas import tpu as pltpu
from jax.experimental.pallas import tpu_sc as plsc

_B = 4096
_D = 1024
_S = 64
_JPAD = 128
_NC, _NS = 2, 16
_NW = _NC * _NS
_RPW = _B // _NW
_G = 16
_NGRP = _RPW // _G
_DC = _D // 16
_NBUF = 4
_UNR = 1


def _sc_scores_body(x_hbm, valid_hbm, samp_hbm, w_hbm, out_hbm,
                    xg2, samp_v2, valid_v2, buf, tbuf, scores_v, sems, sema,
                    semst):
    wid = lax.axis_index("s") * _NC + lax.axis_index("c")
    iota = lax.iota(jnp.int32, 16)

    def reduce16(accs):
        for r in range(16):
            tbuf[pl.ds(r * 16, 16)] = accs[r]
        vec = plsc.load_gather(tbuf, [iota * 16])
        for l in range(1, 16):
            vec = vec + plsc.load_gather(tbuf, [iota * 16 + l])
        return vec

    zeros16 = tuple(jnp.zeros((16,), jnp.float32) for _ in range(16))

    def stage_start(g):
        p = g % 2
        b0 = wid * _RPW + g * _G
        pltpu.async_copy(x_hbm.at[pl.ds(b0, _G)], xg2.at[p], semst)
        pltpu.async_copy(samp_hbm.at[pl.ds(b0 * 4, 64)], samp_v2.at[p], semst)
        pltpu.async_copy(valid_hbm.at[pl.ds(b0, _G)], valid_v2.at[p], semst)

    def stage_wait(g):
        p = g % 2
        b0 = wid * _RPW + g * _G
        pltpu.make_async_copy(x_hbm.at[pl.ds(b0, _G)], xg2.at[p], semst).wait()
        pltpu.make_async_copy(samp_hbm.at[pl.ds(b0 * 4, 64)], samp_v2.at[p],
                              semst).wait()
        pltpu.make_async_copy(valid_hbm.at[pl.ds(b0, _G)], valid_v2.at[p],
                              semst).wait()

    stage_start(0)

    def group(g, _):
        p = g % 2
        xg = xg2.at[p]
        samp_v = samp_v2.at[p]
        valid_v = valid_v2.at[p]
        b0 = wid * _RPW + g * _G
        stage_wait(g)

        pltpu.async_copy(w_hbm.at[valid_v], buf.at[0], sema).wait()

        def start(cc, nb):
            pltpu.async_copy(w_hbm.at[samp_v.at[cc]], buf.at[nb], sems.at[nb])

        for nb in range(1, _NBUF):
            start(nb, nb)

        @pl.when(g + 1 < _NGRP)
        def _():
            stage_start(g + 1)

        def dstep_a(k, accs):
            k16 = k * _UNR * 16
            for u in range(_UNR):
                ku = k16 + u * 16
                accs = tuple(
                    accs[r] + buf[0, r, pl.ds(ku, 16)] * xg[r, pl.ds(ku, 16)]
                    for r in range(16))
            return accs

        s0 = reduce16(lax.fori_loop(0, _DC // _UNR, dstep_a, zeros16))
        plsc.store_scatter(scores_v, [iota * _JPAD], s0)
        start(0, 0)

        def quad(t, _):
            for nb in range(_NBUF):
                c = t * _NBUF + nb
                pltpu.make_async_copy(w_hbm.at[samp_v.at[c]], buf.at[nb],
                                      sems.at[nb]).wait()
                bl = c // 4

                def dstep_b(k, accs, nb=nb, bl=bl):
                    k16 = k * _UNR * 16
                    for u in range(_UNR):
                        ku = k16 + u * 16
                        xc = xg[bl, pl.ds(ku, 16)]
                        accs = tuple(
                            accs[r] + buf[nb, r, pl.ds(ku, 16)] * xc
                            for r in range(16))
                    return accs

                vec = reduce16(lax.fori_loop(0, _DC // _UNR, dstep_b, zeros16))
                off = bl * _JPAD + 1 + (c % 4) * 16
                scores_v[pl.ds(off, 16)] = vec

                @pl.when(c + _NBUF < 64)
                def _(c=c, nb=nb):
                    start(c + _NBUF, nb)
            return 0

        lax.fori_loop(0, 64 // _NBUF, quad, 0)
        pltpu.sync_copy(scores_v, out_hbm.at[pl.ds(b0 * _JPAD, _G * _JPAD)])
        return 0

    lax.fori_loop(0, _NGRP, group, 0)


_sc_scores = functools.partial(
    pl.kernel,
    out_type=jax.ShapeDtypeStruct((_B * _JPAD,), jnp.float32),
    mesh=plsc.VectorSubcoreMesh(core_axis_name="c", subcore_axis_name="s",
                                num_cores=_NC, num_subcores=_NS),
    scratch_types=[
        pltpu.VMEM((2, _G, _D), jnp.float32),
        pltpu.VMEM((2, 64, 16), jnp.int32),
        pltpu.VMEM((2, 16), jnp.int32),
        pltpu.VMEM((_NBUF, 16, _D), jnp.float32),
        pltpu.VMEM((256,), jnp.float32),
        pltpu.VMEM((_G * _JPAD,), jnp.float32),
        pltpu.SemaphoreType.DMA((_NBUF,)),
        pltpu.SemaphoreType.DMA,
        pltpu.SemaphoreType.DMA,
    ],
    compiler_params=pltpu.CompilerParams(needs_layout_passes=False),
)(_sc_scores_body)


def _loss_body(s_ref, o_ref):
    s = s_ref[...]
    lane = lax.broadcasted_iota(jnp.int32, (_B, _JPAD), 1)
    mask = lane < (1 + _S)
    sm = jnp.where(mask, s, -1e30)
    m = jnp.max(sm, axis=1, keepdims=True)
    e = jnp.where(mask, jnp.exp(sm - m), 0.0)
    p = e / jnp.sum(e, axis=1, keepdims=True)
    e2 = jnp.where(mask, jnp.exp(p), 0.0)
    lse = jnp.log(jnp.sum(e2, axis=1, keepdims=True))
    p0 = jnp.sum(jnp.where(lane == 0, p, 0.0), axis=1, keepdims=True)
    o_ref[...] = (jnp.sum(lse - p0) / _B).reshape(1, 1)


def kernel(x, valid_indices, samples_indices, W):
    valid = valid_indices.astype(jnp.int32)
    samp = samples_indices.astype(jnp.int32).reshape(_B * 4, 16)
    scores = _sc_scores(x, valid, samp, W).reshape(_B, _JPAD)
    loss = pl.pallas_call(
        _loss_body,
        out_shape=jax.ShapeDtypeStruct((1, 1), jnp.float32),
    )(scores)
    return loss[0, 0]
